# baseline (speedup 1.0000x reference)
import jax
import jax.numpy as jnp
from jax.experimental import pallas as pl
from jax.experimental.pallas import tpu as pltpu

TOPK = 128
EPS = 0.001
LT = 8


def _body(q_ref, k_ref, v_ref, out_ref, colsum, s_lane, s_row, cut):
    ph = pl.program_id(1)
    t = pl.program_id(2)
    n = s_row.shape[0] // LT
    D = colsum.shape[1]
    L = s_row.shape[0]

    @pl.when(ph == 0)
    def _colsum_phase():
        qp = jax.nn.relu(q_ref[0])
        part = jnp.sum(qp, axis=0, keepdims=True)

        @pl.when(t == 0)
        def _():
            colsum[...] = part

        @pl.when(t != 0)
        def _():
            colsum[...] += part

    @pl.when(ph == 1)
    def _score_phase():
        @pl.when(t == 0)
        def _():
            colsum[...] += jnp.float32(L * EPS)

        kp = jax.nn.relu(k_ref[0])
        cs = colsum[...]
        s0 = EPS * jnp.sum(cs)
        col = jnp.sum(kp * cs, axis=1, keepdims=True) + s0
        s_row[pl.ds(t * n, n), :] = col
        ii = jax.lax.broadcasted_iota(jnp.int32, (1, n), 1)
        jj = jax.lax.broadcasted_iota(jnp.int32, (n, 1), 0)
        idn = (ii == jj).astype(jnp.float32)
        s_lane[pl.ds(t, 1), :] = jax.lax.dot_general(
            col, idn, (((0,), (0,)), ((), ())),
            preferred_element_type=jnp.float32,
            precision=jax.lax.Precision.HIGHEST,
        )

        @pl.when(t == LT - 1)
        def _cutoff():
            sall = s_lane[...]

            def step(_, lohi):
                lo, hi = lohi
                mid = lo + (hi - lo) // 2
                mid_f = jax.lax.bitcast_convert_type(mid, jnp.float32)
                cnt = jnp.sum((sall > mid_f).astype(jnp.int32))
                take = cnt <= TOPK
                return (
                    jnp.where(take, lo, mid + 1),
                    jnp.where(take, mid, hi),
                )

            lo, _ = jax.lax.fori_loop(
                0, 31, step, (jnp.int32(0), jnp.int32(0x7F800000))
            )
            cut[0, 0] = lo

    @pl.when(ph == 2)
    def _mask_phase():
        cut_f = jax.lax.bitcast_convert_type(cut[0, 0], jnp.float32)
        keep = s_row[pl.ds(t * n, n), :] > cut_f
        out_ref[0] = jnp.where(keep, v_ref[0], 0.0)


@jax.jit
def kernel(queries, keys, values):
    B, L, D = queries.shape
    lt_sz = L // LT
    blk = (1, lt_sz, D)

    def q_map(b, ph, t):
        return (b, jnp.where(ph == 0, t, 0), 0)

    def k_map(b, ph, t):
        return (b, jnp.where(ph == 1, t, 0), 0)

    def v_map(b, ph, t):
        return (b, jnp.where(ph == 2, t, 0), 0)

    out = pl.pallas_call(
        _body,
        grid=(B, 3, LT),
        in_specs=[
            pl.BlockSpec(blk, q_map),
            pl.BlockSpec(blk, k_map),
            pl.BlockSpec(blk, v_map),
        ],
        out_specs=pl.BlockSpec(blk, v_map),
        out_shape=jax.ShapeDtypeStruct((B, L, D), jnp.float32),
        scratch_shapes=[
            pltpu.VMEM((1, D), jnp.float32),
            pltpu.VMEM((LT, L // LT), jnp.float32),
            pltpu.VMEM((L, 1), jnp.float32),
            pltpu.SMEM((1, 1), jnp.int32),
        ],
        compiler_params=pltpu.CompilerParams(
            dimension_semantics=("arbitrary", "arbitrary", "arbitrary"),
        ),
    )(queries, keys, values)
    return out
